# baseline (speedup 1.0000x reference)
import jax
import jax.numpy as jnp
from jax import lax
from jax.experimental import pallas as pl
from jax.experimental.pallas import tpu as pltpu
from jax.experimental.pallas import tpu_sc as plsc

N_VOX = 100000
IN_FEAT = 64
K_VOL = 27
CLS_OUT = 4
REG_OUT = 14
D_OUT = 18
D_PAD = 32
PACK = 4

NW = 32
B = 64
N_PAD = 102400
CHUNKS = N_PAD // (NW * B)
PER_W = N_PAD // NW

G = N_PAD // PACK
BG = 1024
NI = G // BG


def _mm_body(x_ref, w_ref, b_ref, y_ref):
    k = pl.program_id(1)
    y_ref[...] = (
        jnp.dot(x_ref[...], w_ref[k], preferred_element_type=jnp.float32)
        + b_ref[k, 0]
    )


def _tc_matmul(x4, w4, b4):
    return pl.pallas_call(
        _mm_body,
        grid=(NI, K_VOL),
        in_specs=[
            pl.BlockSpec((BG, PACK * IN_FEAT), lambda i, k: (i, 0)),
            pl.BlockSpec(
                (K_VOL, PACK * IN_FEAT, PACK * D_PAD), lambda i, k: (0, 0, 0)
            ),
            pl.BlockSpec((K_VOL, 1, PACK * D_PAD), lambda i, k: (0, 0, 0)),
        ],
        out_specs=pl.BlockSpec((BG, PACK * D_PAD), lambda i, k: (k * NI + i, 0)),
        out_shape=jax.ShapeDtypeStruct((K_VOL * G, PACK * D_PAD), jnp.float32),
    )(x4, w4, b4)


def _sc_body(y_hbm, idx_hbm, out_hbm, idx2, buf2, acc_v, sem):
    w = lax.axis_index("s") * 2 + lax.axis_index("c")

    def fire(slot):
        for k in range(K_VOL):
            pltpu.async_copy(
                y_hbm.at[idx2.at[slot].at[k]], buf2.at[slot, k], sem.at[slot]
            )

    def wait_chunk(slot):
        for k in range(K_VOL):
            pltpu.make_async_copy(
                y_hbm.at[pl.ds(0, B)], buf2.at[slot, k], sem.at[slot]
            ).wait()

    def reduce_store(slot, c):
        def red(r, carry2):
            for h in (0, 16):
                v = buf2[slot, 0, r, pl.ds(h, 16)]
                for k in range(1, K_VOL):
                    v = v + buf2[slot, k, r, pl.ds(h, 16)]
                acc_v[r, pl.ds(h, 16)] = v
            return carry2

        lax.fori_loop(0, B, red, 0)
        pltpu.sync_copy(acc_v, out_hbm.at[pl.ds(w * PER_W + c * B, B)])

    pltpu.sync_copy(idx_hbm.at[w, 0], idx2.at[0])
    fire(0)

    def pair(c2, carry):
        for b in (0, 1):
            c = 2 * c2 + b
            nb = 1 - b
            pltpu.sync_copy(idx_hbm.at[w, c + 1], idx2.at[nb])
            fire(nb)
            wait_chunk(b)
            reduce_store(b, c)
        return carry

    lax.fori_loop(0, CHUNKS // 2, pair, 0)
    wait_chunk(0)


def _sc_gather_reduce(y, idx_r):
    mesh = plsc.VectorSubcoreMesh(core_axis_name="c", subcore_axis_name="s")
    fn = pl.kernel(
        _sc_body,
        out_type=jax.ShapeDtypeStruct((N_PAD, D_PAD), jnp.float32),
        mesh=mesh,
        scratch_types=[
            pltpu.VMEM((2, K_VOL, B), jnp.int32),
            pltpu.VMEM((2, K_VOL, B, D_PAD), jnp.float32),
            pltpu.VMEM((B, D_PAD), jnp.float32),
            pltpu.SemaphoreType.DMA((2,)),
        ],
        compiler_params=pltpu.CompilerParams(use_tc_tiling_on_sc=False),
    )
    return fn(y, idx_r)


def kernel(x, neighbor_idx, W_cls, b_cls, W_reg, b_reg):
    x_p = jnp.pad(x, ((0, N_PAD - N_VOX), (0, 0)))
    x4 = x_p.reshape(G, PACK * IN_FEAT)

    w_all = jnp.concatenate([W_cls, W_reg], axis=2)
    w_all = jnp.pad(w_all, ((0, 0), (0, 0), (0, D_PAD - D_OUT)))
    eye4 = jnp.eye(PACK, dtype=jnp.float32)
    w4 = jax.vmap(lambda wk: jnp.kron(eye4, wk))(w_all)

    b_all = jnp.concatenate([b_cls, b_reg])
    b_all = jnp.pad(b_all, (0, D_PAD - D_OUT))
    b4 = jnp.zeros((K_VOL, 1, PACK * D_PAD), jnp.float32)
    b4 = b4.at[0, 0].set(jnp.tile(b_all, PACK))

    idx32 = neighbor_idx.astype(jnp.int32)
    idx_p = jnp.pad(idx32, ((0, N_PAD - N_VOX), (0, 0)))
    flat = idx_p + jnp.arange(K_VOL, dtype=jnp.int32)[None, :] * N_PAD
    idx_r = flat.reshape(NW, CHUNKS, B, K_VOL).transpose(0, 1, 3, 2)
    idx_r = jnp.pad(idx_r, ((0, 0), (0, 1), (0, 0), (0, 0)))

    y4 = _tc_matmul(x4, w4, b4)
    y = y4.reshape(K_VOL * N_PAD, D_PAD)

    out = _sc_gather_reduce(y, idx_r)

    return out[:N_VOX, :CLS_OUT], out[:N_VOX, CLS_OUT:D_OUT]


# speedup vs baseline: 1.3918x; 1.3918x over previous
import jax
import jax.numpy as jnp
from jax import lax
from jax.experimental import pallas as pl
from jax.experimental.pallas import tpu as pltpu
from jax.experimental.pallas import tpu_sc as plsc

N_VOX = 100000
IN_FEAT = 64
K_VOL = 27
CLS_OUT = 4
REG_OUT = 14
D_OUT = 18
D_PAD = 32
PACK = 4

NW = 32
B = 128
N_PAD = 102400
CHUNKS = N_PAD // (NW * B)
PER_W = N_PAD // NW

G = N_PAD // PACK
BG = 1024
NI = G // BG


def _mm_body(x_ref, w_ref, b_ref, y_ref):
    k = pl.program_id(1)
    y_ref[...] = (
        jnp.dot(x_ref[...], w_ref[k], preferred_element_type=jnp.float32)
        + b_ref[k, 0]
    )


def _tc_matmul(x4, w4, b4):
    return pl.pallas_call(
        _mm_body,
        grid=(NI, K_VOL),
        in_specs=[
            pl.BlockSpec((BG, PACK * IN_FEAT), lambda i, k: (i, 0)),
            pl.BlockSpec(
                (K_VOL, PACK * IN_FEAT, PACK * D_PAD), lambda i, k: (0, 0, 0)
            ),
            pl.BlockSpec((K_VOL, 1, PACK * D_PAD), lambda i, k: (0, 0, 0)),
        ],
        out_specs=pl.BlockSpec((BG, PACK * D_PAD), lambda i, k: (k * NI + i, 0)),
        out_shape=jax.ShapeDtypeStruct((K_VOL * G, PACK * D_PAD), jnp.float32),
    )(x4, w4, b4)


def _sc_body(y_hbm, idx_hbm, out_hbm, idx_v, buf_v, acc_v, sem):
    w = lax.axis_index("s") * 2 + lax.axis_index("c")

    def chunk(c, carry):
        pltpu.sync_copy(idx_hbm.at[w, c], idx_v)
        copies = [
            pltpu.async_copy(y_hbm.at[idx_v.at[k]], buf_v.at[k], sem)
            for k in range(K_VOL)
        ]
        for cp in copies:
            cp.wait()

        def red(r, carry2):
            for h in (0, 16):
                v = buf_v[0, r, pl.ds(h, 16)]
                for k in range(1, K_VOL):
                    v = v + buf_v[k, r, pl.ds(h, 16)]
                acc_v[r, pl.ds(h, 16)] = v
            return carry2

        lax.fori_loop(0, B, red, 0)
        pltpu.sync_copy(acc_v, out_hbm.at[pl.ds(w * PER_W + c * B, B)])
        return carry

    lax.fori_loop(0, CHUNKS, chunk, 0)


def _sc_gather_reduce(y, idx_r):
    mesh = plsc.VectorSubcoreMesh(core_axis_name="c", subcore_axis_name="s")
    fn = pl.kernel(
        _sc_body,
        out_type=jax.ShapeDtypeStruct((N_PAD, D_PAD), jnp.float32),
        mesh=mesh,
        scratch_types=[
            pltpu.VMEM((K_VOL, B), jnp.int32),
            pltpu.VMEM((K_VOL, B, D_PAD), jnp.float32),
            pltpu.VMEM((B, D_PAD), jnp.float32),
            pltpu.SemaphoreType.DMA,
        ],
        compiler_params=pltpu.CompilerParams(use_tc_tiling_on_sc=False),
    )
    return fn(y, idx_r)


def kernel(x, neighbor_idx, W_cls, b_cls, W_reg, b_reg):
    x_p = jnp.pad(x, ((0, N_PAD - N_VOX), (0, 0)))
    x4 = x_p.reshape(G, PACK * IN_FEAT)

    w_all = jnp.concatenate([W_cls, W_reg], axis=2)
    w_all = jnp.pad(w_all, ((0, 0), (0, 0), (0, D_PAD - D_OUT)))
    eye4 = jnp.eye(PACK, dtype=jnp.float32)
    w4 = jax.vmap(lambda wk: jnp.kron(eye4, wk))(w_all)

    b_all = jnp.concatenate([b_cls, b_reg])
    b_all = jnp.pad(b_all, (0, D_PAD - D_OUT))
    b4 = jnp.zeros((K_VOL, 1, PACK * D_PAD), jnp.float32)
    b4 = b4.at[0, 0].set(jnp.tile(b_all, PACK))

    idx32 = neighbor_idx.astype(jnp.int32)
    idx_p = jnp.pad(idx32, ((0, N_PAD - N_VOX), (0, 0)))
    flat = idx_p + jnp.arange(K_VOL, dtype=jnp.int32)[None, :] * N_PAD
    idx_r = flat.reshape(NW, CHUNKS, B, K_VOL).transpose(0, 1, 3, 2)

    y4 = _tc_matmul(x4, w4, b4)
    y = y4.reshape(K_VOL * N_PAD, D_PAD)

    out = _sc_gather_reduce(y, idx_r)

    return out[:N_VOX, :CLS_OUT], out[:N_VOX, CLS_OUT:D_OUT]


# speedup vs baseline: 1.7636x; 1.2671x over previous
import jax
import jax.numpy as jnp
from jax import lax
from jax.experimental import pallas as pl
from jax.experimental.pallas import tpu as pltpu
from jax.experimental.pallas import tpu_sc as plsc

N_VOX = 100000
IN_FEAT = 64
K_VOL = 27
CLS_OUT = 4
REG_OUT = 14
D_OUT = 18
D_PAD = 32
DW = 16
PACK = 8

NW = 32
B = 128
N_PAD = 102400
CHUNKS = N_PAD // (NW * B)
PER_W = N_PAD // NW

G = N_PAD // PACK
GV = N_VOX // PACK
BG = 1600
NI = G // BG


def _rne_bf16_hi_bits(f):
    u = jax.lax.bitcast_convert_type(f, jnp.uint32)
    return u + jnp.uint32(0x7FFF) + ((u >> 16) & jnp.uint32(1))


def _mm_body(x_ref, wlo_ref, whi_ref, blo_ref, bhi_ref, y_ref):
    k = pl.program_id(1)
    xb = x_ref[...]
    lo = (
        jnp.dot(xb, wlo_ref[k], preferred_element_type=jnp.float32)
        + blo_ref[k, 0]
    )
    hi = (
        jnp.dot(xb, whi_ref[k], preferred_element_type=jnp.float32)
        + bhi_ref[k, 0]
    )
    word = (_rne_bf16_hi_bits(hi) & jnp.uint32(0xFFFF0000)) | (
        _rne_bf16_hi_bits(lo) >> 16
    )
    y_ref[...] = jax.lax.bitcast_convert_type(word, jnp.float32)


def _tc_matmul(x8, wlo, whi, blo, bhi):
    return pl.pallas_call(
        _mm_body,
        grid=(NI, K_VOL),
        in_specs=[
            pl.BlockSpec((BG, PACK * IN_FEAT), lambda i, k: (i, 0)),
            pl.BlockSpec(
                (K_VOL, PACK * IN_FEAT, PACK * DW), lambda i, k: (0, 0, 0)
            ),
            pl.BlockSpec(
                (K_VOL, PACK * IN_FEAT, PACK * DW), lambda i, k: (0, 0, 0)
            ),
            pl.BlockSpec((K_VOL, 1, PACK * DW), lambda i, k: (0, 0, 0)),
            pl.BlockSpec((K_VOL, 1, PACK * DW), lambda i, k: (0, 0, 0)),
        ],
        out_specs=pl.BlockSpec((BG, PACK * DW), lambda i, k: (k * NI + i, 0)),
        out_shape=jax.ShapeDtypeStruct((K_VOL * G, PACK * DW), jnp.float32),
    )(x8, wlo, whi, blo, bhi)


def _sc_body(y_hbm, idx_hbm, out_hbm, idx_v, buf_v, acc_v, sem):
    w = lax.axis_index("s") * 2 + lax.axis_index("c")

    def chunk(c, carry):
        pltpu.sync_copy(idx_hbm.at[w, c], idx_v)
        copies = [
            pltpu.async_copy(y_hbm.at[idx_v.at[k]], buf_v.at[k], sem)
            for k in range(K_VOL)
        ]
        for cp in copies:
            cp.wait()

        def red(r, carry2):
            a0, b0 = plsc.unpack(
                plsc.bitcast(buf_v[0, r, :], jnp.bfloat16),
                format=plsc.PackFormat.INTERLEAVED,
            )
            for k in range(1, K_VOL):
                a, b = plsc.unpack(
                    plsc.bitcast(buf_v[k, r, :], jnp.bfloat16),
                    format=plsc.PackFormat.INTERLEAVED,
                )
                a0 = a0 + a
                b0 = b0 + b
            acc_v[r, pl.ds(0, 16)] = a0
            acc_v[r, pl.ds(16, 16)] = b0
            return carry2

        lax.fori_loop(0, B, red, 0)
        pltpu.sync_copy(acc_v, out_hbm.at[pl.ds(w * PER_W + c * B, B)])
        return carry

    lax.fori_loop(0, CHUNKS, chunk, 0)


def _sc_gather_reduce(y, idx_r):
    mesh = plsc.VectorSubcoreMesh(core_axis_name="c", subcore_axis_name="s")
    fn = pl.kernel(
        _sc_body,
        out_type=jax.ShapeDtypeStruct((N_PAD, D_PAD), jnp.float32),
        mesh=mesh,
        scratch_types=[
            pltpu.VMEM((K_VOL, B), jnp.int32),
            pltpu.VMEM((K_VOL, B, DW), jnp.float32),
            pltpu.VMEM((B, D_PAD), jnp.float32),
            pltpu.SemaphoreType.DMA,
        ],
        compiler_params=pltpu.CompilerParams(
            use_tc_tiling_on_sc=False, needs_layout_passes=False
        ),
    )
    return fn(y, idx_r)


def kernel(x, neighbor_idx, W_cls, b_cls, W_reg, b_reg):
    x8 = x.astype(jnp.bfloat16).reshape(GV, PACK * IN_FEAT)

    w_all = jnp.concatenate([W_cls, W_reg], axis=2)
    w_all = jnp.pad(w_all, ((0, 0), (0, 0), (0, D_PAD - D_OUT)))
    w_lo = w_all[:, :, :DW].astype(jnp.bfloat16)
    w_hi = w_all[:, :, DW:].astype(jnp.bfloat16)
    eye8 = jnp.eye(PACK, dtype=jnp.bfloat16)
    wlo = jax.vmap(lambda wk: jnp.kron(eye8, wk))(w_lo)
    whi = jax.vmap(lambda wk: jnp.kron(eye8, wk))(w_hi)

    b_all = jnp.concatenate([b_cls, b_reg])
    b_all = jnp.pad(b_all, (0, D_PAD - D_OUT))
    blo = jnp.zeros((K_VOL, 1, PACK * DW), jnp.float32)
    blo = blo.at[0, 0].set(jnp.tile(b_all[:DW], PACK))
    bhi = jnp.zeros((K_VOL, 1, PACK * DW), jnp.float32)
    bhi = bhi.at[0, 0].set(jnp.tile(b_all[DW:], PACK))

    idx32 = neighbor_idx.astype(jnp.int32)
    idx_p = jnp.pad(idx32, ((0, N_PAD - N_VOX), (0, 0)))
    flat = idx_p + jnp.arange(K_VOL, dtype=jnp.int32)[None, :] * N_PAD
    idx_r = flat.reshape(NW, CHUNKS, B, K_VOL).transpose(0, 1, 3, 2)

    y8 = _tc_matmul(x8, wlo, whi, blo, bhi)
    y = y8.reshape(K_VOL * N_PAD, DW)

    out = _sc_gather_reduce(y, idx_r)

    return out[:N_VOX, :CLS_OUT], out[:N_VOX, CLS_OUT:D_OUT]


# speedup vs baseline: 1.7941x; 1.0173x over previous
import jax
import jax.numpy as jnp
from jax import lax
from jax.experimental import pallas as pl
from jax.experimental.pallas import tpu as pltpu
from jax.experimental.pallas import tpu_sc as plsc

N_VOX = 100000
IN_FEAT = 64
K_VOL = 27
CLS_OUT = 4
REG_OUT = 14
D_OUT = 18
D_PAD = 32
DW = 16
PACK = 8

NW = 32
B = 128
N_PAD = 102400
CHUNKS = N_PAD // (NW * B)
PER_W = N_PAD // NW

G = N_PAD // PACK
GV = N_VOX // PACK
BG = 1600
NI = G // BG


def _rne_bf16_hi_bits(f):
    u = jax.lax.bitcast_convert_type(f, jnp.uint32)
    return u + jnp.uint32(0x7FFF) + ((u >> 16) & jnp.uint32(1))


def _mm_body(x_ref, wlo_ref, whi_ref, blo_ref, bhi_ref, y_ref):
    k = pl.program_id(1)
    xb = x_ref[...]
    lo = (
        jnp.dot(xb, wlo_ref[k], preferred_element_type=jnp.float32)
        + blo_ref[k, 0]
    )
    hi = (
        jnp.dot(xb, whi_ref[k], preferred_element_type=jnp.float32)
        + bhi_ref[k, 0]
    )
    word = (_rne_bf16_hi_bits(hi) & jnp.uint32(0xFFFF0000)) | (
        _rne_bf16_hi_bits(lo) >> 16
    )
    y_ref[...] = jax.lax.bitcast_convert_type(word, jnp.float32)


def _tc_matmul(x8, wlo, whi, blo, bhi):
    return pl.pallas_call(
        _mm_body,
        grid=(NI, K_VOL),
        in_specs=[
            pl.BlockSpec((BG, PACK * IN_FEAT), lambda i, k: (i, 0)),
            pl.BlockSpec(
                (K_VOL, PACK * IN_FEAT, PACK * DW), lambda i, k: (0, 0, 0)
            ),
            pl.BlockSpec(
                (K_VOL, PACK * IN_FEAT, PACK * DW), lambda i, k: (0, 0, 0)
            ),
            pl.BlockSpec((K_VOL, 1, PACK * DW), lambda i, k: (0, 0, 0)),
            pl.BlockSpec((K_VOL, 1, PACK * DW), lambda i, k: (0, 0, 0)),
        ],
        out_specs=pl.BlockSpec((BG, PACK * DW), lambda i, k: (k * NI + i, 0)),
        out_shape=jax.ShapeDtypeStruct((K_VOL * G, PACK * DW), jnp.float32),
    )(x8, wlo, whi, blo, bhi)


def _sc_body(y_hbm, idx_hbm, out_hbm, idx_v, buf_v, acc_v, sem):
    w = lax.axis_index("s") * 2 + lax.axis_index("c")

    KA = 14

    def chunk(c, carry):
        pltpu.sync_copy(idx_hbm.at[w, c], idx_v)
        copies = [
            pltpu.async_copy(
                y_hbm.at[idx_v.at[k]], buf_v.at[k], sem.at[0 if k < KA else 1]
            )
            for k in range(K_VOL)
        ]

        def unpack_k(k, r):
            return plsc.unpack(
                plsc.bitcast(buf_v[k, r, :], jnp.bfloat16),
                format=plsc.PackFormat.INTERLEAVED,
            )

        for cp in copies[:KA]:
            cp.wait()

        def red_a(r, carry2):
            a0, b0 = unpack_k(0, r)
            for k in range(1, KA):
                a, b = unpack_k(k, r)
                a0 = a0 + a
                b0 = b0 + b
            acc_v[r, pl.ds(0, 16)] = a0
            acc_v[r, pl.ds(16, 16)] = b0
            return carry2

        lax.fori_loop(0, B, red_a, 0)
        for cp in copies[KA:]:
            cp.wait()

        def red_b(r, carry2):
            a0, b0 = unpack_k(KA, r)
            for k in range(KA + 1, K_VOL):
                a, b = unpack_k(k, r)
                a0 = a0 + a
                b0 = b0 + b
            acc_v[r, pl.ds(0, 16)] = acc_v[r, pl.ds(0, 16)] + a0
            acc_v[r, pl.ds(16, 16)] = acc_v[r, pl.ds(16, 16)] + b0
            return carry2

        lax.fori_loop(0, B, red_b, 0)
        pltpu.sync_copy(acc_v, out_hbm.at[pl.ds(w * PER_W + c * B, B)])
        return carry

    lax.fori_loop(0, CHUNKS, chunk, 0)


def _sc_gather_reduce(y, idx_r):
    mesh = plsc.VectorSubcoreMesh(core_axis_name="c", subcore_axis_name="s")
    fn = pl.kernel(
        _sc_body,
        out_type=jax.ShapeDtypeStruct((N_PAD, D_PAD), jnp.float32),
        mesh=mesh,
        scratch_types=[
            pltpu.VMEM((K_VOL, B), jnp.int32),
            pltpu.VMEM((K_VOL, B, DW), jnp.float32),
            pltpu.VMEM((B, D_PAD), jnp.float32),
            pltpu.SemaphoreType.DMA((2,)),
        ],
        compiler_params=pltpu.CompilerParams(
            use_tc_tiling_on_sc=False, needs_layout_passes=False
        ),
    )
    return fn(y, idx_r)


def kernel(x, neighbor_idx, W_cls, b_cls, W_reg, b_reg):
    x8 = x.astype(jnp.bfloat16).reshape(GV, PACK * IN_FEAT)

    w_all = jnp.concatenate([W_cls, W_reg], axis=2)
    w_all = jnp.pad(w_all, ((0, 0), (0, 0), (0, D_PAD - D_OUT)))
    w_lo = w_all[:, :, :DW].astype(jnp.bfloat16)
    w_hi = w_all[:, :, DW:].astype(jnp.bfloat16)
    eye8 = jnp.eye(PACK, dtype=jnp.bfloat16)
    wlo = jax.vmap(lambda wk: jnp.kron(eye8, wk))(w_lo)
    whi = jax.vmap(lambda wk: jnp.kron(eye8, wk))(w_hi)

    b_all = jnp.concatenate([b_cls, b_reg])
    b_all = jnp.pad(b_all, (0, D_PAD - D_OUT))
    blo = jnp.zeros((K_VOL, 1, PACK * DW), jnp.float32)
    blo = blo.at[0, 0].set(jnp.tile(b_all[:DW], PACK))
    bhi = jnp.zeros((K_VOL, 1, PACK * DW), jnp.float32)
    bhi = bhi.at[0, 0].set(jnp.tile(b_all[DW:], PACK))

    idx32 = neighbor_idx.astype(jnp.int32)
    idx_p = jnp.pad(idx32, ((0, N_PAD - N_VOX), (0, 0)))
    flat = idx_p + jnp.arange(K_VOL, dtype=jnp.int32)[None, :] * N_PAD
    idx_r = flat.reshape(NW, CHUNKS, B, K_VOL).transpose(0, 1, 3, 2)

    y8 = _tc_matmul(x8, wlo, whi, blo, bhi)
    y = y8.reshape(K_VOL * N_PAD, DW)

    out = _sc_gather_reduce(y, idx_r)

    return out[:N_VOX, :CLS_OUT], out[:N_VOX, CLS_OUT:D_OUT]


# speedup vs baseline: 1.9279x; 1.0746x over previous
import jax
import jax.numpy as jnp
from jax import lax
from jax.experimental import pallas as pl
from jax.experimental.pallas import tpu as pltpu
from jax.experimental.pallas import tpu_sc as plsc

N_VOX = 100000
IN_FEAT = 64
K_VOL = 27
K_A = 14
K_B = K_VOL - K_A
CLS_OUT = 4
REG_OUT = 14
D_OUT = 18
D_PAD = 32
DW = 16
PACK = 8

NW = 32
B = 128
N_PAD = 102400
CHUNKS = N_PAD // (NW * B)
PER_W = N_PAD // NW

G = N_PAD // PACK
GV = N_VOX // PACK
BG = 1600
NI = G // BG


def _rne_bf16_hi_bits(f):
    u = jax.lax.bitcast_convert_type(f, jnp.uint32)
    return u + jnp.uint32(0x7FFF) + ((u >> 16) & jnp.uint32(1))


def _mm_body(x_ref, wlo_ref, whi_ref, blo_ref, bhi_ref, y_ref):
    k = pl.program_id(1)
    xb = x_ref[...]
    lo = (
        jnp.dot(xb, wlo_ref[k], preferred_element_type=jnp.float32)
        + blo_ref[k, 0]
    )
    hi = (
        jnp.dot(xb, whi_ref[k], preferred_element_type=jnp.float32)
        + bhi_ref[k, 0]
    )
    word = (_rne_bf16_hi_bits(hi) & jnp.uint32(0xFFFF0000)) | (
        _rne_bf16_hi_bits(lo) >> 16
    )
    y_ref[...] = jax.lax.bitcast_convert_type(word, jnp.float32)


def _tc_matmul(x8, wlo, whi, blo, bhi, kvol):
    return pl.pallas_call(
        _mm_body,
        grid=(NI, kvol),
        in_specs=[
            pl.BlockSpec((BG, PACK * IN_FEAT), lambda i, k: (i, 0)),
            pl.BlockSpec(
                (kvol, PACK * IN_FEAT, PACK * DW), lambda i, k: (0, 0, 0)
            ),
            pl.BlockSpec(
                (kvol, PACK * IN_FEAT, PACK * DW), lambda i, k: (0, 0, 0)
            ),
            pl.BlockSpec((kvol, 1, PACK * DW), lambda i, k: (0, 0, 0)),
            pl.BlockSpec((kvol, 1, PACK * DW), lambda i, k: (0, 0, 0)),
        ],
        out_specs=pl.BlockSpec((BG, PACK * DW), lambda i, k: (k * NI + i, 0)),
        out_shape=jax.ShapeDtypeStruct((kvol * G, PACK * DW), jnp.float32),
    )(x8, wlo, whi, blo, bhi)


def _make_sc_body(kvol, with_prev):
    ka = kvol // 2

    def body(y_hbm, idx_hbm, *rest):
        if with_prev:
            prev_hbm, out_hbm, idx_v, buf_v, acc_v, prev_v, sem = rest
        else:
            out_hbm, idx_v, buf_v, acc_v, sem = rest
        w = lax.axis_index("s") * 2 + lax.axis_index("c")

        def chunk(c, carry):
            base = w * PER_W + c * B
            pltpu.sync_copy(idx_hbm.at[w, c], idx_v)
            copies = [
                pltpu.async_copy(
                    y_hbm.at[idx_v.at[k]],
                    buf_v.at[k],
                    sem.at[0 if k < ka else 1],
                )
                for k in range(kvol)
            ]
            if with_prev:
                pltpu.sync_copy(prev_hbm.at[pl.ds(base, B)], prev_v)

            def unpack_k(k, r):
                return plsc.unpack(
                    plsc.bitcast(buf_v[k, r, :], jnp.bfloat16),
                    format=plsc.PackFormat.INTERLEAVED,
                )

            for cp in copies[:ka]:
                cp.wait()

            def red_a(r, carry2):
                a0, b0 = unpack_k(0, r)
                for k in range(1, ka):
                    a, b = unpack_k(k, r)
                    a0 = a0 + a
                    b0 = b0 + b
                if with_prev:
                    a0 = a0 + prev_v[r, pl.ds(0, 16)]
                    b0 = b0 + prev_v[r, pl.ds(16, 16)]
                acc_v[r, pl.ds(0, 16)] = a0
                acc_v[r, pl.ds(16, 16)] = b0
                return carry2

            lax.fori_loop(0, B, red_a, 0)
            for cp in copies[ka:]:
                cp.wait()

            def red_b(r, carry2):
                a0, b0 = unpack_k(ka, r)
                for k in range(ka + 1, kvol):
                    a, b = unpack_k(k, r)
                    a0 = a0 + a
                    b0 = b0 + b
                acc_v[r, pl.ds(0, 16)] = acc_v[r, pl.ds(0, 16)] + a0
                acc_v[r, pl.ds(16, 16)] = acc_v[r, pl.ds(16, 16)] + b0
                return carry2

            lax.fori_loop(0, B, red_b, 0)
            pltpu.sync_copy(acc_v, out_hbm.at[pl.ds(base, B)])
            return carry

        lax.fori_loop(0, CHUNKS, chunk, 0)

    return body


def _sc_gather_reduce(y, idx_r, kvol, prev=None):
    mesh = plsc.VectorSubcoreMesh(core_axis_name="c", subcore_axis_name="s")
    scratch = [
        pltpu.VMEM((kvol, B), jnp.int32),
        pltpu.VMEM((kvol, B, DW), jnp.float32),
        pltpu.VMEM((B, D_PAD), jnp.float32),
    ]
    if prev is not None:
        scratch.append(pltpu.VMEM((B, D_PAD), jnp.float32))
    scratch.append(pltpu.SemaphoreType.DMA((2,)))
    fn = pl.kernel(
        _make_sc_body(kvol, prev is not None),
        out_type=jax.ShapeDtypeStruct((N_PAD, D_PAD), jnp.float32),
        mesh=mesh,
        scratch_types=scratch,
        compiler_params=pltpu.CompilerParams(
            use_tc_tiling_on_sc=False, needs_layout_passes=False
        ),
    )
    if prev is not None:
        return fn(y, idx_r, prev)
    return fn(y, idx_r)


def _pack_weights(w_sub, b_sub, with_bias):
    kvol = w_sub.shape[0]
    w_lo = w_sub[:, :, :DW].astype(jnp.bfloat16)
    w_hi = w_sub[:, :, DW:].astype(jnp.bfloat16)
    eye8 = jnp.eye(PACK, dtype=jnp.bfloat16)
    wlo = jax.vmap(lambda wk: jnp.kron(eye8, wk))(w_lo)
    whi = jax.vmap(lambda wk: jnp.kron(eye8, wk))(w_hi)
    blo = jnp.zeros((kvol, 1, PACK * DW), jnp.float32)
    bhi = jnp.zeros((kvol, 1, PACK * DW), jnp.float32)
    if with_bias:
        blo = blo.at[0, 0].set(jnp.tile(b_sub[:DW], PACK))
        bhi = bhi.at[0, 0].set(jnp.tile(b_sub[DW:], PACK))
    return wlo, whi, blo, bhi


def kernel(x, neighbor_idx, W_cls, b_cls, W_reg, b_reg):
    x8 = x.astype(jnp.bfloat16).reshape(GV, PACK * IN_FEAT)

    w_all = jnp.concatenate([W_cls, W_reg], axis=2)
    w_all = jnp.pad(w_all, ((0, 0), (0, 0), (0, D_PAD - D_OUT)))
    b_all = jnp.concatenate([b_cls, b_reg])
    b_all = jnp.pad(b_all, (0, D_PAD - D_OUT))
    wlo_a, whi_a, blo_a, bhi_a = _pack_weights(w_all[:K_A], b_all, True)
    wlo_b, whi_b, blo_b, bhi_b = _pack_weights(w_all[K_A:], b_all, False)

    idx32 = neighbor_idx.astype(jnp.int32)
    idx_p = jnp.pad(idx32, ((0, N_PAD - N_VOX), (0, 0)))
    karange = jnp.arange(K_VOL, dtype=jnp.int32) % jnp.int32(K_A)
    flat = idx_p + karange[None, :] * N_PAD
    idx_ra = (
        flat[:, :K_A].reshape(NW, CHUNKS, B, K_A).transpose(0, 1, 3, 2)
    )
    idx_rb = (
        flat[:, K_A:].reshape(NW, CHUNKS, B, K_B).transpose(0, 1, 3, 2)
    )

    ya = _tc_matmul(x8, wlo_a, whi_a, blo_a, bhi_a, K_A)
    yb = _tc_matmul(x8, wlo_b, whi_b, blo_b, bhi_b, K_B)

    out_a = _sc_gather_reduce(ya.reshape(K_A * N_PAD, DW), idx_ra, K_A)
    out = _sc_gather_reduce(
        yb.reshape(K_B * N_PAD, DW), idx_rb, K_B, prev=out_a
    )

    return out[:N_VOX, :CLS_OUT], out[:N_VOX, CLS_OUT:D_OUT]
